# Initial kernel scaffold; baseline (speedup 1.0000x reference)
import functools

import jax
import jax.numpy as jnp
from jax import lax
from jax.experimental import pallas as pl
from jax.experimental.pallas import tpu as pltpu
from jax.experimental.pallas import tpu_sc as plsc

_K = 5
_N = 10000
_E = 160000
_H1 = 32
_H2 = 16
_NG = 64

_NC = 2
_NS = 16
_NW = _NC * _NS

_N_PAD = 10112
_ROWS_SUB = _N_PAD // _NS
_E_BLK = 128
_E_TILE = 5120
_E_PAD = _E_TILE * _NW

_mesh = functools.partial(
    plsc.VectorSubcoreMesh,
    core_axis_name="c", subcore_axis_name="s",
    num_cores=_NC, num_subcores=_NS,
)


def _worker():
    cid = lax.axis_index("c")
    sid = lax.axis_index("s")
    return cid, sid, cid * _NS + sid


def _deg_body(src_hbm, w_hbm, out_hbm, wbuf, sbuf, zbuf, deg_sh):
    cid, sid, wid = _worker()

    for i in range(_ROWS_SUB // 16):
        zbuf[pl.ds(i * 16, 16)] = jnp.zeros((16,), jnp.float32)
    pltpu.sync_copy(zbuf, deg_sh.at[pl.ds(sid * _ROWS_SUB, _ROWS_SUB)])
    plsc.subcore_barrier()

    base = wid * _E_TILE

    def chunk(j, _):
        eb = base + j * _E_BLK
        pltpu.sync_copy(src_hbm.at[pl.ds(eb, _E_BLK)], sbuf.at[0])
        pltpu.sync_copy(w_hbm.at[pl.ds(eb, _E_BLK)], wbuf)
        pltpu.sync_copy(wbuf, deg_sh.at[sbuf.at[0]], add=True)
        return 0

    lax.fori_loop(0, _E_TILE // _E_BLK, chunk, 0)
    plsc.subcore_barrier()
    pltpu.sync_copy(deg_sh.at[pl.ds(sid * _ROWS_SUB, _ROWS_SUB)],
                    out_hbm.at[cid, pl.ds(sid * _ROWS_SUB, _ROWS_SUB)])


_deg_kernel = pl.kernel(
    _deg_body,
    out_type=jax.ShapeDtypeStruct((_NC, _N_PAD), jnp.float32),
    mesh=_mesh(),
    scratch_types=[
        pltpu.VMEM((_E_BLK,), jnp.float32),
        pltpu.VMEM((1, _E_BLK), jnp.int32),
        pltpu.VMEM((_ROWS_SUB,), jnp.float32),
        pltpu.VMEM_SHARED((_N_PAD,), jnp.float32),
    ],
)


def _norm_body(dis_hbm, src_hbm, dst_hbm, w_hbm, out_hbm,
               disbuf, sbuf, dbuf, wbuf, nbuf):
    _, _, wid = _worker()
    pltpu.sync_copy(dis_hbm, disbuf)
    base = wid * _E_TILE
    pltpu.sync_copy(src_hbm.at[pl.ds(base, _E_TILE)], sbuf)
    pltpu.sync_copy(dst_hbm.at[pl.ds(base, _E_TILE)], dbuf)
    pltpu.sync_copy(w_hbm.at[pl.ds(base, _E_TILE)], wbuf)

    def step(i, _):
        sv = sbuf[pl.ds(i * 16, 16)]
        dv = dbuf[pl.ds(i * 16, 16)]
        wv = wbuf[pl.ds(i * 16, 16)]
        a = plsc.load_gather(disbuf, [sv])
        b = plsc.load_gather(disbuf, [dv])
        nbuf[pl.ds(i * 16, 16)] = -a * wv * b
        return 0

    lax.fori_loop(0, _E_TILE // 16, step, 0)
    pltpu.sync_copy(nbuf, out_hbm.at[pl.ds(base, _E_TILE)])


_norm_kernel = pl.kernel(
    _norm_body,
    out_type=jax.ShapeDtypeStruct((_E_PAD,), jnp.float32),
    mesh=_mesh(),
    scratch_types=[
        pltpu.VMEM((_N_PAD,), jnp.float32),
        pltpu.VMEM((_E_TILE,), jnp.int32),
        pltpu.VMEM((_E_TILE,), jnp.int32),
        pltpu.VMEM((_E_TILE,), jnp.float32),
        pltpu.VMEM((_E_TILE,), jnp.float32),
    ],
)


def _prop_body(d, t_hbm, src_hbm, dst_hbm, norm_hbm, out_hbm,
               sbuf, dbuf, nbuf, rows, zbuf, y_sh, sem):
    cid, sid, wid = _worker()
    nvec = d // 16

    zr = zbuf.shape[0]
    for i in range(zr):
        for c in range(nvec):
            zbuf[i, pl.ds(c * 16, 16)] = jnp.zeros((16,), jnp.float32)
    for r in range(_ROWS_SUB // zr):
        pltpu.sync_copy(zbuf, y_sh.at[pl.ds(sid * _ROWS_SUB + r * zr, zr)])
    plsc.subcore_barrier()

    base = wid * _E_TILE

    def chunk(j, _):
        eb = base + j * _E_BLK
        pltpu.sync_copy(src_hbm.at[pl.ds(eb, _E_BLK)], sbuf.at[0])
        pltpu.sync_copy(dst_hbm.at[pl.ds(eb, _E_BLK)], dbuf.at[0])
        pltpu.sync_copy(norm_hbm.at[pl.ds(eb, _E_BLK)], nbuf)
        pltpu.async_copy(t_hbm.at[sbuf.at[0]], rows, sem).wait()

        def scale(e, _):
            sn = nbuf[e]
            for c in range(nvec):
                rows[e, pl.ds(c * 16, 16)] = rows[e, pl.ds(c * 16, 16)] * sn
            return 0

        lax.fori_loop(0, _E_BLK, scale, 0)
        pltpu.sync_copy(rows, y_sh.at[dbuf.at[0]], add=True)
        return 0

    lax.fori_loop(0, _E_TILE // _E_BLK, chunk, 0)
    plsc.subcore_barrier()
    pltpu.sync_copy(y_sh.at[pl.ds(sid * _ROWS_SUB, _ROWS_SUB)],
                    out_hbm.at[cid, pl.ds(sid * _ROWS_SUB, _ROWS_SUB)])


@functools.cache
def _prop_kernel(d):
    zr = 158
    return pl.kernel(
        functools.partial(_prop_body, d),
        out_type=jax.ShapeDtypeStruct((_NC, _N_PAD, d), jnp.float32),
        mesh=_mesh(),
        scratch_types=[
            pltpu.VMEM((1, _E_BLK), jnp.int32),
            pltpu.VMEM((1, _E_BLK), jnp.int32),
            pltpu.VMEM((_E_BLK,), jnp.float32),
            pltpu.VMEM((_E_BLK, d), jnp.float32),
            pltpu.VMEM((zr, d), jnp.float32),
            pltpu.VMEM_SHARED((_N_PAD, d), jnp.float32),
            pltpu.SemaphoreType.DMA,
        ],
    )


_RB = _N_PAD // 8


def _dis_body(degp_ref, out_ref):
    deg = degp_ref[0:1, :] + degp_ref[1:2, :]
    out_ref[...] = jnp.where(
        deg > 0.0, lax.rsqrt(jnp.where(deg > 0.0, deg, 1.0)), 0.0)


def _dis_call(degp):
    return pl.pallas_call(
        _dis_body,
        out_shape=jax.ShapeDtypeStruct((1, _N_PAD), jnp.float32),
    )(degp)


def _combine_body(has_prev, y_ref, *rest):
    if has_prev:
        p_ref, out_ref = rest
        out_ref[...] = 2.0 * (y_ref[0] + y_ref[1]) - p_ref[...]
    else:
        (out_ref,) = rest
        out_ref[...] = y_ref[0] + y_ref[1]


@functools.cache
def _combine_call(d, has_prev):
    specs = [pl.BlockSpec((_NC, _RB, d), lambda i: (0, i, 0))]
    if has_prev:
        specs.append(pl.BlockSpec((_RB, d), lambda i: (i, 0)))
    return pl.pallas_call(
        functools.partial(_combine_body, has_prev),
        grid=(8,),
        in_specs=specs,
        out_specs=pl.BlockSpec((_RB, d), lambda i: (i, 0)),
        out_shape=jax.ShapeDtypeStruct((_N_PAD, d), jnp.float32),
    )


def _gates_body(oc, b_ref, w_ref, bias_ref, wc_ref, c_ref, h_out, c_out):
    bias = (bias_ref[0:1, :] + bias_ref[1:2, :] + bias_ref[2:3, :])
    z = jnp.broadcast_to(bias, (b_ref.shape[1], 4 * oc)).astype(jnp.float32)
    for k in range(_K):
        z = z + jnp.dot(b_ref[k], w_ref[k], preferred_element_type=jnp.float32)
    c_prev = c_ref[...]
    i_g = jax.nn.sigmoid(z[:, 0 * oc:1 * oc] + wc_ref[0:1, :] * c_prev)
    f_g = jax.nn.sigmoid(z[:, 1 * oc:2 * oc] + wc_ref[1:2, :] * c_prev)
    t_g = jnp.tanh(z[:, 2 * oc:3 * oc])
    c_new = f_g * c_prev + i_g * t_g
    o_g = jax.nn.sigmoid(z[:, 3 * oc:4 * oc] + wc_ref[2:3, :] * c_new)
    h_out[...] = o_g * jnp.tanh(c_new)
    c_out[...] = c_new


@functools.cache
def _gates_call(d, oc):
    return pl.pallas_call(
        functools.partial(_gates_body, oc),
        grid=(8,),
        in_specs=[
            pl.BlockSpec((_K, _RB, d), lambda i: (0, i, 0)),
            pl.BlockSpec((_K, d, 4 * oc), lambda i: (0, 0, 0)),
            pl.BlockSpec((3, 4 * oc), lambda i: (0, 0)),
            pl.BlockSpec((3, oc), lambda i: (0, 0)),
            pl.BlockSpec((_RB, oc), lambda i: (i, 0)),
        ],
        out_specs=[
            pl.BlockSpec((_RB, oc), lambda i: (i, 0)),
            pl.BlockSpec((_RB, oc), lambda i: (i, 0)),
        ],
        out_shape=[
            jax.ShapeDtypeStruct((_N_PAD, oc), jnp.float32),
            jax.ShapeDtypeStruct((_N_PAD, oc), jnp.float32),
        ],
    )


def _pool_body(h2_ref, b_ref, lw_ref, lb_ref, out_ref, u_scr):
    i = pl.program_id(0)

    @pl.when(i == 0)
    def _():
        u_scr[...] = jnp.zeros_like(u_scr)

    oh = (b_ref[...] == lax.broadcasted_iota(jnp.int32, (_RB, _NG), 1))
    oh = oh.astype(jnp.float32)
    u_scr[...] += lax.dot_general(
        oh, h2_ref[...], (((0,), (0,)), ((), ())),
        preferred_element_type=jnp.float32)

    @pl.when(i == pl.num_programs(0) - 1)
    def _():
        out_ref[...] = (jnp.dot(u_scr[...], lw_ref[...],
                                preferred_element_type=jnp.float32)
                        + lb_ref[...])


_pool_call = pl.pallas_call(
    _pool_body,
    grid=(8,),
    in_specs=[
        pl.BlockSpec((_RB, _H2), lambda i: (i, 0)),
        pl.BlockSpec((_RB, 1), lambda i: (i, 0)),
        pl.BlockSpec((_H2, 1), lambda i: (0, 0)),
        pl.BlockSpec((1, 1), lambda i: (0, 0)),
    ],
    out_specs=pl.BlockSpec((_NG, 1), lambda i: (0, 0)),
    out_shape=jax.ShapeDtypeStruct((_NG, 1), jnp.float32),
    scratch_shapes=[pltpu.VMEM((_NG, _H2), jnp.float32)],
)


def _basis(table, src_p, dst_p, norm_p):
    d = table.shape[1]
    prop = _prop_kernel(d)
    comb0 = _combine_call(d, False)
    comb1 = _combine_call(d, True)
    txs = [table]
    y = prop(table, src_p, dst_p, norm_p)
    txs.append(comb0(y))
    for _ in range(2, _K):
        y = prop(txs[-1], src_p, dst_p, norm_p)
        txs.append(comb1(y, txs[-2]))
    return jnp.stack(txs)


def _cell(pcat, table, src_p, dst_p, norm_p, c_prev, x_only):
    wcat, bias3, wc = pcat
    d = table.shape[1]
    oc = wc.shape[1]
    if x_only:
        wcat = wcat[:, :d, :]
    b = _basis(table, src_p, dst_p, norm_p)
    h, c = _gates_call(d, oc)(b, wcat, bias3, wc, c_prev)
    return h, c


def _prep_cell_params(p):
    wx = jnp.concatenate([p['Wx_' + g] for g in 'ifco'], axis=2)
    wh = jnp.concatenate([p['Wh_' + g] for g in 'ifco'], axis=2)
    wcat = jnp.concatenate([wx, wh], axis=1)
    bias3 = jnp.stack([
        jnp.concatenate([p['bx_' + g] for g in 'ifco']),
        jnp.concatenate([p['bh_' + g] for g in 'ifco']),
        jnp.concatenate([p['b_' + g] for g in 'ifco']),
    ])
    wc = jnp.stack([p['w_c_i'], p['w_c_f'], p['w_c_o']])
    return wcat, bias3, wc


def kernel(x, edge_index, edge_weight, batch, params):
    t_steps = x.shape[0]
    src = edge_index[0].astype(jnp.int32)
    dst = edge_index[1].astype(jnp.int32)

    npad_e = _E_PAD - _E
    src_p = jnp.concatenate([src, jnp.zeros((npad_e,), jnp.int32)])
    dst_p = jnp.concatenate(
        [dst, _N + (jnp.arange(npad_e, dtype=jnp.int32) % (_N_PAD - _N))])
    w_p = jnp.concatenate([edge_weight, jnp.zeros((npad_e,), jnp.float32)])

    xp = jnp.pad(x, ((0, 0), (0, _N_PAD - _N), (0, 0)))
    batch_p = jnp.concatenate(
        [batch.astype(jnp.int32), jnp.full((_N_PAD - _N,), _NG, jnp.int32)])

    degp = _deg_kernel(src_p, w_p)
    dis = _dis_call(degp).reshape(-1)
    norm_p = _norm_kernel(dis, src_p, dst_p, w_p)

    p1 = _prep_cell_params(params['l1'])
    p2 = _prep_cell_params(params['l2'])

    c1 = jnp.zeros((_N_PAD, _H1), jnp.float32)
    c2 = jnp.zeros((_N_PAD, _H2), jnp.float32)
    h1 = h2 = None
    for t in range(t_steps):
        tbl1 = xp[t] if t == 0 else jnp.concatenate([xp[t], h1], axis=1)
        h1, c1 = _cell(p1, tbl1, src_p, dst_p, norm_p, c1, x_only=(t == 0))
        tbl2 = h1 if t == 0 else jnp.concatenate([h1, h2], axis=1)
        h2, c2 = _cell(p2, tbl2, src_p, dst_p, norm_p, c2, x_only=(t == 0))

    out = _pool_call(h2, batch_p.reshape(_N_PAD, 1),
                     params['lin_W'], params['lin_b'].reshape(1, 1))
    return out.reshape(-1)


# speedup vs baseline: 3.2358x; 3.2358x over previous
import functools

import jax
import jax.numpy as jnp
from jax import lax
from jax.experimental import pallas as pl
from jax.experimental.pallas import tpu as pltpu
from jax.experimental.pallas import tpu_sc as plsc

_K = 5
_N = 10000
_E = 160000
_H1 = 32
_H2 = 16
_NG = 64

_NC = 2
_NS = 16
_NW = _NC * _NS

_N_PAD = 10112
_ROWS_SUB = _N_PAD // _NS
_E_BLK = 128
_E_TILE = 5120
_E_PAD = _E_TILE * _NW
_NH = _N_PAD // 2
_YR = _NH + 64
_RSUB = _YR // _NS
_NCHUNKS = _E_PAD // _E_BLK

_mesh = functools.partial(
    plsc.VectorSubcoreMesh,
    core_axis_name="c", subcore_axis_name="s",
    num_cores=_NC, num_subcores=_NS,
)
_sc_params = pltpu.CompilerParams(use_tc_tiling_on_sc=False)


def _worker():
    cid = lax.axis_index("c")
    sid = lax.axis_index("s")
    return cid, sid, cid * _NS + sid


def _deg_body(src_hbm, w_hbm, out_hbm, wbuf, sbuf, rows, zbuf, deg_sh):
    cid, sid, wid = _worker()

    def zrow(i, _):
        zbuf[i, pl.ds(0, 16)] = jnp.zeros((16,), jnp.float32)
        return 0

    lax.fori_loop(0, _ROWS_SUB, zrow, 0)
    pltpu.sync_copy(zbuf, deg_sh.at[pl.ds(sid * _ROWS_SUB, _ROWS_SUB)])
    plsc.subcore_barrier()

    base = wid * _E_TILE

    def chunk(j, _):
        eb = base + j * _E_BLK
        pltpu.sync_copy(src_hbm.at[pl.ds(eb, _E_BLK)], sbuf.at[0])
        pltpu.sync_copy(w_hbm.at[pl.ds(eb, _E_BLK)], wbuf)

        def build(g, _):
            nv = wbuf[pl.ds(g * 16, 16)]
            for l in range(16):
                rows[g * 16 + l, pl.ds(0, 16)] = jnp.full(
                    (16,), nv[l], jnp.float32)
            return 0

        lax.fori_loop(0, _E_BLK // 16, build, 0)
        pltpu.sync_copy(rows, deg_sh.at[sbuf.at[0]], add=True)
        return 0

    lax.fori_loop(0, _E_TILE // _E_BLK, chunk, 0)
    plsc.subcore_barrier()
    pltpu.sync_copy(deg_sh.at[pl.ds(sid * _ROWS_SUB, _ROWS_SUB)], zbuf)
    pltpu.sync_copy(
        zbuf, out_hbm.at[pl.ds(cid * _N_PAD + sid * _ROWS_SUB, _ROWS_SUB)])


_deg_kernel = pl.kernel(
    _deg_body,
    out_type=jax.ShapeDtypeStruct((_NC * _N_PAD, 16), jnp.float32),
    mesh=_mesh(),
    scratch_types=[
        pltpu.VMEM((_E_BLK,), jnp.float32),
        pltpu.VMEM((1, _E_BLK), jnp.int32),
        pltpu.VMEM((_E_BLK, 16), jnp.float32),
        pltpu.VMEM((_ROWS_SUB, 16), jnp.float32),
        pltpu.VMEM_SHARED((_N_PAD, 16), jnp.float32),
    ],
    compiler_params=_sc_params,
)


def _norm_body(dis_hbm, src_hbm, dst_hbm, w_hbm, out_hbm,
               sbuf, dbuf, wbuf, av, bv, nbuf, sem):
    _, _, wid = _worker()
    base = wid * _E_TILE

    def chunk(j, _):
        eb = base + j * _E_BLK
        pltpu.sync_copy(src_hbm.at[pl.ds(eb, _E_BLK)], sbuf.at[0])
        pltpu.sync_copy(dst_hbm.at[pl.ds(eb, _E_BLK)], dbuf.at[0])
        pltpu.sync_copy(w_hbm.at[pl.ds(eb, _E_BLK)], wbuf)
        pltpu.async_copy(dis_hbm.at[sbuf.at[0]], av, sem).wait()
        pltpu.async_copy(dis_hbm.at[dbuf.at[0]], bv, sem).wait()
        for i in range(_E_BLK // 16):
            s = pl.ds(i * 16, 16)
            nbuf[s] = -av[s] * wbuf[s] * bv[s]
        pltpu.sync_copy(nbuf, out_hbm.at[pl.ds(eb, _E_BLK)])
        return 0

    lax.fori_loop(0, _E_TILE // _E_BLK, chunk, 0)


_norm_kernel = pl.kernel(
    _norm_body,
    out_type=jax.ShapeDtypeStruct((_E_PAD,), jnp.float32),
    mesh=_mesh(),
    scratch_types=[
        pltpu.VMEM((1, _E_BLK), jnp.int32),
        pltpu.VMEM((1, _E_BLK), jnp.int32),
        pltpu.VMEM((_E_BLK,), jnp.float32),
        pltpu.VMEM((_E_BLK,), jnp.float32),
        pltpu.VMEM((_E_BLK,), jnp.float32),
        pltpu.VMEM((_E_BLK,), jnp.float32),
        pltpu.SemaphoreType.DMA,
    ],
    compiler_params=_sc_params,
)


def _prop_body(d, t_hbm, src_hbm, dst_hbm, norm_hbm, cnts_hbm, out_hbm,
               sbuf, dbuf, dbuf2, nbuf, cbuf, rows, zbuf, y_sh, sem):
    cid, sid, wid = _worker()
    nvec = d // 16

    def zrow(i, _):
        for c in range(nvec):
            zbuf[i, pl.ds(c * 16, 16)] = jnp.zeros((16,), jnp.float32)
        return 0

    lax.fori_loop(0, _RSUB, zrow, 0)
    pltpu.sync_copy(zbuf, y_sh.at[pl.ds(sid * _RSUB, _RSUB)])
    plsc.subcore_barrier()

    pltpu.sync_copy(cnts_hbm.at[pl.ds(cid * 16, 16)], cbuf)
    cv = cbuf[pl.ds(0, 16)]
    start = cv[0]
    end = cv[1]
    nk = jnp.maximum(0, (end - start - sid + _NS - 1) // _NS)
    base_row = cid * _NH

    def chunk(k, _):
        eb = (start + sid + k * _NS) * _E_BLK
        pltpu.sync_copy(src_hbm.at[pl.ds(eb, _E_BLK)], sbuf.at[0])
        pltpu.sync_copy(dst_hbm.at[pl.ds(eb, _E_BLK)], dbuf.at[0])
        pltpu.sync_copy(norm_hbm.at[pl.ds(eb, _E_BLK)], nbuf)
        pltpu.async_copy(t_hbm.at[sbuf.at[0]], rows, sem).wait()

        for g in range(_E_BLK // 16):
            dl = dbuf[0, pl.ds(g * 16, 16)] - base_row
            ok = (dl >= 0) & (dl < _NH)
            dummy = _NH + lax.iota(jnp.int32, 16) + (g % 4) * 16
            dbuf2[0, pl.ds(g * 16, 16)] = jnp.where(ok, dl, dummy)

        def scale(g, _):
            nv = nbuf[pl.ds(g * 16, 16)]
            for l in range(16):
                sn = nv[l]
                e = g * 16 + l
                for c in range(nvec):
                    rows[e, pl.ds(c * 16, 16)] = rows[e, pl.ds(c * 16, 16)] * sn
            return 0

        lax.fori_loop(0, _E_BLK // 16, scale, 0)
        pltpu.sync_copy(rows, y_sh.at[dbuf2.at[0]], add=True)
        return 0

    lax.fori_loop(0, nk, chunk, 0)
    plsc.subcore_barrier()

    @pl.when(sid < _NS - 1)
    def _():
        r0 = sid * _RSUB
        pltpu.sync_copy(y_sh.at[pl.ds(r0, _RSUB)], zbuf)
        pltpu.sync_copy(zbuf, out_hbm.at[pl.ds(base_row + r0, _RSUB)])

    @pl.when(sid == _NS - 1)
    def _():
        r0 = (_NS - 1) * _RSUB
        nlast = _NH - r0
        pltpu.sync_copy(y_sh.at[pl.ds(r0, nlast)], zbuf.at[pl.ds(0, nlast)])
        pltpu.sync_copy(zbuf.at[pl.ds(0, nlast)],
                        out_hbm.at[pl.ds(base_row + r0, nlast)])


@functools.cache
def _prop_kernel(d):
    return pl.kernel(
        functools.partial(_prop_body, d),
        out_type=jax.ShapeDtypeStruct((_N_PAD, d), jnp.float32),
        mesh=_mesh(),
        scratch_types=[
            pltpu.VMEM((1, _E_BLK), jnp.int32),
            pltpu.VMEM((1, _E_BLK), jnp.int32),
            pltpu.VMEM((1, _E_BLK), jnp.int32),
            pltpu.VMEM((_E_BLK,), jnp.float32),
            pltpu.VMEM((16,), jnp.int32),
            pltpu.VMEM((_E_BLK, d), jnp.float32),
            pltpu.VMEM((_RSUB, d), jnp.float32),
            pltpu.VMEM_SHARED((_YR, d), jnp.float32),
            pltpu.SemaphoreType.DMA,
        ],
        compiler_params=_sc_params,
    )


_RB = _N_PAD // 8


def _dis_body(degp_ref, out_ref):
    deg = degp_ref[0:_N_PAD, 0:1] + degp_ref[_N_PAD:2 * _N_PAD, 0:1]
    out_ref[...] = jnp.where(
        deg > 0.0, lax.rsqrt(jnp.where(deg > 0.0, deg, 1.0)), 0.0)


def _dis_call(degp):
    return pl.pallas_call(
        _dis_body,
        out_shape=jax.ShapeDtypeStruct((_N_PAD, 1), jnp.float32),
    )(degp)


def _combine_body(y_ref, p_ref, out_ref):
    out_ref[...] = 2.0 * y_ref[...] - p_ref[...]


@functools.cache
def _combine_call(d):
    spec = pl.BlockSpec((_RB, d), lambda i: (i, 0))
    return pl.pallas_call(
        _combine_body,
        grid=(8,),
        in_specs=[spec, spec],
        out_specs=spec,
        out_shape=jax.ShapeDtypeStruct((_N_PAD, d), jnp.float32),
    )


def _gates_body(oc, b_ref, w_ref, bias_ref, wc_ref, c_ref, h_out, c_out):
    bias = (bias_ref[0:1, :] + bias_ref[1:2, :] + bias_ref[2:3, :])
    z = jnp.broadcast_to(bias, (b_ref.shape[1], 4 * oc)).astype(jnp.float32)
    for k in range(_K):
        z = z + jnp.dot(b_ref[k], w_ref[k], preferred_element_type=jnp.float32)
    c_prev = c_ref[...]
    i_g = jax.nn.sigmoid(z[:, 0 * oc:1 * oc] + wc_ref[0:1, :] * c_prev)
    f_g = jax.nn.sigmoid(z[:, 1 * oc:2 * oc] + wc_ref[1:2, :] * c_prev)
    t_g = jnp.tanh(z[:, 2 * oc:3 * oc])
    c_new = f_g * c_prev + i_g * t_g
    o_g = jax.nn.sigmoid(z[:, 3 * oc:4 * oc] + wc_ref[2:3, :] * c_new)
    h_out[...] = o_g * jnp.tanh(c_new)
    c_out[...] = c_new


@functools.cache
def _gates_call(d, oc):
    return pl.pallas_call(
        functools.partial(_gates_body, oc),
        grid=(8,),
        in_specs=[
            pl.BlockSpec((_K, _RB, d), lambda i: (0, i, 0)),
            pl.BlockSpec((_K, d, 4 * oc), lambda i: (0, 0, 0)),
            pl.BlockSpec((3, 4 * oc), lambda i: (0, 0)),
            pl.BlockSpec((3, oc), lambda i: (0, 0)),
            pl.BlockSpec((_RB, oc), lambda i: (i, 0)),
        ],
        out_specs=[
            pl.BlockSpec((_RB, oc), lambda i: (i, 0)),
            pl.BlockSpec((_RB, oc), lambda i: (i, 0)),
        ],
        out_shape=[
            jax.ShapeDtypeStruct((_N_PAD, oc), jnp.float32),
            jax.ShapeDtypeStruct((_N_PAD, oc), jnp.float32),
        ],
    )


def _pool_body(h2_ref, b_ref, lw_ref, lb_ref, out_ref, u_scr):
    i = pl.program_id(0)

    @pl.when(i == 0)
    def _():
        u_scr[...] = jnp.zeros_like(u_scr)

    oh = (b_ref[...] == lax.broadcasted_iota(jnp.int32, (_RB, _NG), 1))
    oh = oh.astype(jnp.float32)
    u_scr[...] += lax.dot_general(
        oh, h2_ref[...], (((0,), (0,)), ((), ())),
        preferred_element_type=jnp.float32)

    @pl.when(i == pl.num_programs(0) - 1)
    def _():
        out_ref[...] = (jnp.dot(u_scr[...], lw_ref[...],
                                preferred_element_type=jnp.float32)
                        + lb_ref[...])


_pool_call = pl.pallas_call(
    _pool_body,
    grid=(8,),
    in_specs=[
        pl.BlockSpec((_RB, _H2), lambda i: (i, 0)),
        pl.BlockSpec((_RB, 1), lambda i: (i, 0)),
        pl.BlockSpec((_H2, 1), lambda i: (0, 0)),
        pl.BlockSpec((1, 1), lambda i: (0, 0)),
    ],
    out_specs=pl.BlockSpec((_NG, 1), lambda i: (0, 0)),
    out_shape=jax.ShapeDtypeStruct((_NG, 1), jnp.float32),
    scratch_shapes=[pltpu.VMEM((_NG, _H2), jnp.float32)],
)


def _basis(table, src_p, dst_p, norm_p, cnts):
    d = table.shape[1]
    prop = _prop_kernel(d)
    comb = _combine_call(d)
    txs = [table, prop(table, src_p, dst_p, norm_p, cnts)]
    for _ in range(2, _K):
        y = prop(txs[-1], src_p, dst_p, norm_p, cnts)
        txs.append(comb(y, txs[-2]))
    return jnp.stack(txs)


def _cell(pcat, table, edges, c_prev, x_only):
    wcat, bias3, wc = pcat
    d = table.shape[1]
    oc = wc.shape[1]
    if x_only:
        wcat = wcat[:, :d, :]
    b = _basis(table, *edges)
    h, c = _gates_call(d, oc)(b, wcat, bias3, wc, c_prev)
    return h, c


def _prep_cell_params(p):
    wx = jnp.concatenate([p['Wx_' + g] for g in 'ifco'], axis=2)
    wh = jnp.concatenate([p['Wh_' + g] for g in 'ifco'], axis=2)
    wcat = jnp.concatenate([wx, wh], axis=1)
    bias3 = jnp.stack([
        jnp.concatenate([p['bx_' + g] for g in 'ifco']),
        jnp.concatenate([p['bh_' + g] for g in 'ifco']),
        jnp.concatenate([p['b_' + g] for g in 'ifco']),
    ])
    wc = jnp.stack([p['w_c_i'], p['w_c_f'], p['w_c_o']])
    return wcat, bias3, wc


def kernel(x, edge_index, edge_weight, batch, params):
    t_steps = x.shape[0]
    src = edge_index[0].astype(jnp.int32)
    dst = edge_index[1].astype(jnp.int32)

    order = jnp.argsort((dst >= _NH).astype(jnp.int32), stable=True)
    cnt0 = jnp.sum((dst < _NH).astype(jnp.int32))
    npad_e = _E_PAD - _E
    src_p = jnp.concatenate([src[order], jnp.zeros((npad_e,), jnp.int32)])
    dst_p = jnp.concatenate(
        [dst[order], jnp.full((npad_e,), 2 * _N, jnp.int32)])
    w_p = jnp.concatenate(
        [edge_weight[order], jnp.zeros((npad_e,), jnp.float32)])
    zi = jnp.zeros((14,), jnp.int32)
    cnts = jnp.concatenate([
        jnp.stack([jnp.int32(0), (cnt0 + _E_BLK - 1) // _E_BLK]), zi,
        jnp.stack([cnt0 // _E_BLK, jnp.int32(_NCHUNKS)]), zi,
    ])

    xp = jnp.pad(x, ((0, 0), (0, _N_PAD - _N), (0, 0)))
    batch_p = jnp.concatenate(
        [batch.astype(jnp.int32), jnp.full((_N_PAD - _N,), _NG, jnp.int32)])

    degp = _deg_kernel(src_p, w_p)
    dis = _dis_call(degp).reshape(-1)
    norm_p = _norm_kernel(dis, src_p, dst_p, w_p)

    p1 = _prep_cell_params(params['l1'])
    p2 = _prep_cell_params(params['l2'])

    edges = (src_p, dst_p, norm_p, cnts)
    c1 = jnp.zeros((_N_PAD, _H1), jnp.float32)
    c2 = jnp.zeros((_N_PAD, _H2), jnp.float32)
    h1 = h2 = None
    for t in range(t_steps):
        tbl1 = xp[t] if t == 0 else jnp.concatenate([xp[t], h1], axis=1)
        h1, c1 = _cell(p1, tbl1, edges, c1, x_only=(t == 0))
        tbl2 = h1 if t == 0 else jnp.concatenate([h1, h2], axis=1)
        h2, c2 = _cell(p2, tbl2, edges, c2, x_only=(t == 0))

    out = _pool_call(h2, batch_p.reshape(_N_PAD, 1),
                     params['lin_W'], params['lin_b'].reshape(1, 1))
    return out.reshape(-1)


# speedup vs baseline: 3.3325x; 1.0299x over previous
import functools

import jax
import jax.numpy as jnp
from jax import lax
from jax.experimental import pallas as pl
from jax.experimental.pallas import tpu as pltpu
from jax.experimental.pallas import tpu_sc as plsc

_K = 5
_N = 10000
_E = 160000
_H1 = 32
_H2 = 16
_NG = 64

_NC = 2
_NS = 16
_NW = _NC * _NS

_N_PAD = 10112
_ROWS_SUB = _N_PAD // _NS
_E_BLK = 128
_E_TILE = 5120
_E_PAD = _E_TILE * _NW
_NH = _N_PAD // 2
_YR = _NH + 64
_RSUB = _YR // _NS
_NCHUNKS = _E_PAD // _E_BLK

_mesh = functools.partial(
    plsc.VectorSubcoreMesh,
    core_axis_name="c", subcore_axis_name="s",
    num_cores=_NC, num_subcores=_NS,
)
_sc_params = pltpu.CompilerParams(use_tc_tiling_on_sc=False)


def _worker():
    cid = lax.axis_index("c")
    sid = lax.axis_index("s")
    return cid, sid, cid * _NS + sid


def _deg_body(src_hbm, w_hbm, out_hbm, wbuf, sbuf, rows, zbuf, deg_sh):
    cid, sid, wid = _worker()

    def zrow(i, _):
        zbuf[i, pl.ds(0, 16)] = jnp.zeros((16,), jnp.float32)
        return 0

    lax.fori_loop(0, _ROWS_SUB, zrow, 0)
    pltpu.sync_copy(zbuf, deg_sh.at[pl.ds(sid * _ROWS_SUB, _ROWS_SUB)])
    plsc.subcore_barrier()

    base = wid * _E_TILE

    def chunk(j, _):
        eb = base + j * _E_BLK
        pltpu.sync_copy(src_hbm.at[pl.ds(eb, _E_BLK)], sbuf.at[0])
        pltpu.sync_copy(w_hbm.at[pl.ds(eb, _E_BLK)], wbuf)

        def build(g, _):
            nv = wbuf[pl.ds(g * 16, 16)]
            for l in range(16):
                rows[g * 16 + l, pl.ds(0, 16)] = jnp.full(
                    (16,), nv[l], jnp.float32)
            return 0

        lax.fori_loop(0, _E_BLK // 16, build, 0)
        pltpu.sync_copy(rows, deg_sh.at[sbuf.at[0]], add=True)
        return 0

    lax.fori_loop(0, _E_TILE // _E_BLK, chunk, 0)
    plsc.subcore_barrier()
    pltpu.sync_copy(deg_sh.at[pl.ds(sid * _ROWS_SUB, _ROWS_SUB)], zbuf)
    pltpu.sync_copy(
        zbuf, out_hbm.at[pl.ds(cid * _N_PAD + sid * _ROWS_SUB, _ROWS_SUB)])


_deg_kernel = pl.kernel(
    _deg_body,
    out_type=jax.ShapeDtypeStruct((_NC * _N_PAD, 16), jnp.float32),
    mesh=_mesh(),
    scratch_types=[
        pltpu.VMEM((_E_BLK,), jnp.float32),
        pltpu.VMEM((1, _E_BLK), jnp.int32),
        pltpu.VMEM((_E_BLK, 16), jnp.float32),
        pltpu.VMEM((_ROWS_SUB, 16), jnp.float32),
        pltpu.VMEM_SHARED((_N_PAD, 16), jnp.float32),
    ],
    compiler_params=_sc_params,
)


def _norm_body(dis_hbm, src_hbm, dst_hbm, w_hbm, out_hbm,
               sbuf, dbuf, wbuf, av, bv, nbuf, sem):
    _, _, wid = _worker()
    base = wid * _E_TILE

    def chunk(j, _):
        eb = base + j * _E_BLK
        pltpu.sync_copy(src_hbm.at[pl.ds(eb, _E_BLK)], sbuf.at[0])
        pltpu.sync_copy(dst_hbm.at[pl.ds(eb, _E_BLK)], dbuf.at[0])
        pltpu.sync_copy(w_hbm.at[pl.ds(eb, _E_BLK)], wbuf)
        pltpu.async_copy(dis_hbm.at[sbuf.at[0]], av, sem).wait()
        pltpu.async_copy(dis_hbm.at[dbuf.at[0]], bv, sem).wait()
        for i in range(_E_BLK // 16):
            s = pl.ds(i * 16, 16)
            nbuf[s] = -av[s] * wbuf[s] * bv[s]
        pltpu.sync_copy(nbuf, out_hbm.at[pl.ds(eb, _E_BLK)])
        return 0

    lax.fori_loop(0, _E_TILE // _E_BLK, chunk, 0)


_norm_kernel = pl.kernel(
    _norm_body,
    out_type=jax.ShapeDtypeStruct((_E_PAD,), jnp.float32),
    mesh=_mesh(),
    scratch_types=[
        pltpu.VMEM((1, _E_BLK), jnp.int32),
        pltpu.VMEM((1, _E_BLK), jnp.int32),
        pltpu.VMEM((_E_BLK,), jnp.float32),
        pltpu.VMEM((_E_BLK,), jnp.float32),
        pltpu.VMEM((_E_BLK,), jnp.float32),
        pltpu.VMEM((_E_BLK,), jnp.float32),
        pltpu.SemaphoreType.DMA,
    ],
    compiler_params=_sc_params,
)


_EW = 2 * _E_BLK


def _prop_body(d, t_hbm, edata_hbm, norm_hbm, cnts_hbm, out_hbm,
               ebuf, nbuf, dbuf2, cbuf, rows, zbuf, y_sh, sem_i, sem_g, sem_s):
    cid, sid, wid = _worker()
    nvec = d // 16

    def zrow(i, _):
        for c in range(nvec):
            zbuf[i, pl.ds(c * 16, 16)] = jnp.zeros((16,), jnp.float32)
        return 0

    lax.fori_loop(0, _RSUB, zrow, 0)
    pltpu.sync_copy(zbuf, y_sh.at[pl.ds(sid * _RSUB, _RSUB)])
    plsc.subcore_barrier()

    pltpu.sync_copy(cnts_hbm.at[pl.ds(cid * 16, 16)], cbuf)
    cv = cbuf[pl.ds(0, 16)]
    start = cv[0]
    end = cv[1]
    nk = jnp.maximum(0, (end - start - sid + _NS - 1) // _NS)
    base_row = cid * _NH

    def eoff(k):
        return (start + sid + k * _NS) * _EW

    def noff(k):
        return (start + sid + k * _NS) * _E_BLK

    def body(k, _):
        slot = lax.rem(k, 2)
        pslot = lax.rem(k + 1, 2)
        e3c = lax.rem(k + 1, 3)
        e3p = lax.rem(k + 2, 3)
        e3i = lax.rem(k, 3)

        @pl.when((k >= 3) & (k - 3 < nk))
        def _():
            pltpu.make_async_copy(
                rows.at[pslot], y_sh.at[dbuf2.at[pslot]], sem_s.at[pslot]
            ).wait()

        @pl.when((k >= 1) & (k <= nk))
        def _():
            pltpu.make_async_copy(
                edata_hbm.at[pl.ds(0, _EW)], ebuf.at[e3p], sem_i).wait()
            pltpu.make_async_copy(
                norm_hbm.at[pl.ds(0, _E_BLK)], nbuf.at[e3p], sem_i).wait()
            pltpu.async_copy(
                t_hbm.at[ebuf.at[e3p, pl.ds(0, _E_BLK)]], rows.at[pslot],
                sem_g.at[pslot])

        @pl.when(k < nk)
        def _():
            pltpu.async_copy(
                edata_hbm.at[pl.ds(eoff(k), _EW)], ebuf.at[e3i], sem_i)
            pltpu.async_copy(
                norm_hbm.at[pl.ds(noff(k), _E_BLK)], nbuf.at[e3i], sem_i)

        @pl.when(k >= 2)
        def _():
            pltpu.make_async_copy(
                t_hbm.at[ebuf.at[e3c, pl.ds(0, _E_BLK)]], rows.at[slot],
                sem_g.at[slot]).wait()

            for g in range(_E_BLK // 16):
                dl = ebuf[e3c, pl.ds(_E_BLK + g * 16, 16)] - base_row
                ok = (dl >= 0) & (dl < _NH)
                dummy = _NH + lax.iota(jnp.int32, 16) + (g % 4) * 16
                dbuf2[slot, pl.ds(g * 16, 16)] = jnp.where(ok, dl, dummy)

            def scale(g, _):
                nv = nbuf[e3c, pl.ds(g * 16, 16)]
                for l in range(16):
                    sn = nv[l]
                    e = g * 16 + l
                    for c in range(nvec):
                        rows[slot, e, pl.ds(c * 16, 16)] = (
                            rows[slot, e, pl.ds(c * 16, 16)] * sn)
                return 0

            lax.fori_loop(0, _E_BLK // 16, scale, 0)
            pltpu.async_copy(
                rows.at[slot], y_sh.at[dbuf2.at[slot]], sem_s.at[slot],
                add=True)

        return 0

    lax.fori_loop(0, nk + 2, body, 0)

    @pl.when(nk >= 1)
    def _():
        lslot = lax.rem(nk + 1, 2)
        pltpu.make_async_copy(
            rows.at[lslot], y_sh.at[dbuf2.at[lslot]], sem_s.at[lslot]).wait()

    plsc.subcore_barrier()

    @pl.when(sid < _NS - 1)
    def _():
        r0 = sid * _RSUB
        pltpu.sync_copy(y_sh.at[pl.ds(r0, _RSUB)], zbuf)
        pltpu.sync_copy(zbuf, out_hbm.at[pl.ds(base_row + r0, _RSUB)])

    @pl.when(sid == _NS - 1)
    def _():
        r0 = (_NS - 1) * _RSUB
        nlast = _NH - r0
        pltpu.sync_copy(y_sh.at[pl.ds(r0, nlast)], zbuf.at[pl.ds(0, nlast)])
        pltpu.sync_copy(zbuf.at[pl.ds(0, nlast)],
                        out_hbm.at[pl.ds(base_row + r0, nlast)])


@functools.cache
def _prop_kernel(d):
    return pl.kernel(
        functools.partial(_prop_body, d),
        out_type=jax.ShapeDtypeStruct((_N_PAD, d), jnp.float32),
        mesh=_mesh(),
        scratch_types=[
            pltpu.VMEM((3, _EW), jnp.int32),
            pltpu.VMEM((3, _E_BLK), jnp.float32),
            pltpu.VMEM((2, _E_BLK), jnp.int32),
            pltpu.VMEM((16,), jnp.int32),
            pltpu.VMEM((2, _E_BLK, d), jnp.float32),
            pltpu.VMEM((_RSUB, d), jnp.float32),
            pltpu.VMEM_SHARED((_YR, d), jnp.float32),
            pltpu.SemaphoreType.DMA,
            pltpu.SemaphoreType.DMA((2,)),
            pltpu.SemaphoreType.DMA((2,)),
        ],
        compiler_params=_sc_params,
    )


_RB = _N_PAD // 8


def _dis_body(degp_ref, out_ref):
    deg = degp_ref[0:_N_PAD, 0:1] + degp_ref[_N_PAD:2 * _N_PAD, 0:1]
    out_ref[...] = jnp.where(
        deg > 0.0, lax.rsqrt(jnp.where(deg > 0.0, deg, 1.0)), 0.0)


def _dis_call(degp):
    return pl.pallas_call(
        _dis_body,
        out_shape=jax.ShapeDtypeStruct((_N_PAD, 1), jnp.float32),
    )(degp)


def _combine_body(y_ref, p_ref, out_ref):
    out_ref[...] = 2.0 * y_ref[...] - p_ref[...]


@functools.cache
def _combine_call(d):
    spec = pl.BlockSpec((_RB, d), lambda i: (i, 0))
    return pl.pallas_call(
        _combine_body,
        grid=(8,),
        in_specs=[spec, spec],
        out_specs=spec,
        out_shape=jax.ShapeDtypeStruct((_N_PAD, d), jnp.float32),
    )


def _gates_body(oc, nb, *refs):
    b_refs = refs[:nb]
    w_refs = refs[nb:2 * nb]
    bias_ref, wc_ref, c_ref, h_out, c_out = refs[2 * nb:]
    bias = (bias_ref[0:1, :] + bias_ref[1:2, :] + bias_ref[2:3, :])
    z = jnp.broadcast_to(bias, (_RB, 4 * oc)).astype(jnp.float32)
    for b_ref, w_ref in zip(b_refs, w_refs):
        for k in range(_K):
            z = z + jnp.dot(b_ref[k], w_ref[k],
                            preferred_element_type=jnp.float32)
    c_prev = c_ref[...]
    i_g = jax.nn.sigmoid(z[:, 0 * oc:1 * oc] + wc_ref[0:1, :] * c_prev)
    f_g = jax.nn.sigmoid(z[:, 1 * oc:2 * oc] + wc_ref[1:2, :] * c_prev)
    t_g = jnp.tanh(z[:, 2 * oc:3 * oc])
    c_new = f_g * c_prev + i_g * t_g
    o_g = jax.nn.sigmoid(z[:, 3 * oc:4 * oc] + wc_ref[2:3, :] * c_new)
    h_out[...] = o_g * jnp.tanh(c_new)
    c_out[...] = c_new


@functools.cache
def _gates_call(ds, oc):
    return pl.pallas_call(
        functools.partial(_gates_body, oc, len(ds)),
        grid=(8,),
        in_specs=(
            [pl.BlockSpec((_K, _RB, d), lambda i: (0, i, 0)) for d in ds]
            + [pl.BlockSpec((_K, d, 4 * oc), lambda i: (0, 0, 0)) for d in ds]
            + [
                pl.BlockSpec((3, 4 * oc), lambda i: (0, 0)),
                pl.BlockSpec((3, oc), lambda i: (0, 0)),
                pl.BlockSpec((_RB, oc), lambda i: (i, 0)),
            ]
        ),
        out_specs=[
            pl.BlockSpec((_RB, oc), lambda i: (i, 0)),
            pl.BlockSpec((_RB, oc), lambda i: (i, 0)),
        ],
        out_shape=[
            jax.ShapeDtypeStruct((_N_PAD, oc), jnp.float32),
            jax.ShapeDtypeStruct((_N_PAD, oc), jnp.float32),
        ],
    )


def _pool_body(h2_ref, b_ref, lw_ref, lb_ref, out_ref, u_scr):
    i = pl.program_id(0)

    @pl.when(i == 0)
    def _():
        u_scr[...] = jnp.zeros_like(u_scr)

    oh = (b_ref[...] == lax.broadcasted_iota(jnp.int32, (_RB, _NG), 1))
    oh = oh.astype(jnp.float32)
    u_scr[...] += lax.dot_general(
        oh, h2_ref[...], (((0,), (0,)), ((), ())),
        preferred_element_type=jnp.float32)

    @pl.when(i == pl.num_programs(0) - 1)
    def _():
        out_ref[...] = (jnp.dot(u_scr[...], lw_ref[...],
                                preferred_element_type=jnp.float32)
                        + lb_ref[...])


_pool_call = pl.pallas_call(
    _pool_body,
    grid=(8,),
    in_specs=[
        pl.BlockSpec((_RB, _H2), lambda i: (i, 0)),
        pl.BlockSpec((_RB, 1), lambda i: (i, 0)),
        pl.BlockSpec((_H2, 1), lambda i: (0, 0)),
        pl.BlockSpec((1, 1), lambda i: (0, 0)),
    ],
    out_specs=pl.BlockSpec((_NG, 1), lambda i: (0, 0)),
    out_shape=jax.ShapeDtypeStruct((_NG, 1), jnp.float32),
    scratch_shapes=[pltpu.VMEM((_NG, _H2), jnp.float32)],
)


def _basis(table, edata, norm_p, cnts):
    d = table.shape[1]
    prop = _prop_kernel(d)
    comb = _combine_call(d)
    txs = [table, prop(table, edata, norm_p, cnts)]
    for _ in range(2, _K):
        y = prop(txs[-1], edata, norm_p, cnts)
        txs.append(comb(y, txs[-2]))
    return jnp.stack(txs)


def _cell(pcat, tables, edges, c_prev):
    wcat, bias3, wc = pcat
    oc = wc.shape[1]
    bs, ws, off = [], [], 0
    for t in tables:
        d = t.shape[1]
        bs.append(_basis(t, *edges))
        ws.append(wcat[:, off:off + d, :])
        off += d
    ds = tuple(t.shape[1] for t in tables)
    h, c = _gates_call(ds, oc)(*bs, *ws, bias3, wc, c_prev)
    return h, c


def _prep_cell_params(p):
    wx = jnp.concatenate([p['Wx_' + g] for g in 'ifco'], axis=2)
    wh = jnp.concatenate([p['Wh_' + g] for g in 'ifco'], axis=2)
    wcat = jnp.concatenate([wx, wh], axis=1)
    bias3 = jnp.stack([
        jnp.concatenate([p['bx_' + g] for g in 'ifco']),
        jnp.concatenate([p['bh_' + g] for g in 'ifco']),
        jnp.concatenate([p['b_' + g] for g in 'ifco']),
    ])
    wc = jnp.stack([p['w_c_i'], p['w_c_f'], p['w_c_o']])
    return wcat, bias3, wc


def kernel(x, edge_index, edge_weight, batch, params):
    t_steps = x.shape[0]
    src = edge_index[0].astype(jnp.int32)
    dst = edge_index[1].astype(jnp.int32)

    order = jnp.argsort((dst >= _NH).astype(jnp.int32), stable=True)
    cnt0 = jnp.sum((dst < _NH).astype(jnp.int32))
    npad_e = _E_PAD - _E
    src_p = jnp.concatenate([src[order], jnp.zeros((npad_e,), jnp.int32)])
    dst_p = jnp.concatenate(
        [dst[order], jnp.full((npad_e,), 2 * _N, jnp.int32)])
    w_p = jnp.concatenate(
        [edge_weight[order], jnp.zeros((npad_e,), jnp.float32)])
    zi = jnp.zeros((14,), jnp.int32)
    cnts = jnp.concatenate([
        jnp.stack([jnp.int32(0), (cnt0 + _E_BLK - 1) // _E_BLK]), zi,
        jnp.stack([cnt0 // _E_BLK, jnp.int32(_NCHUNKS)]), zi,
    ])

    xp = jnp.pad(x, ((0, 0), (0, _N_PAD - _N), (0, 0)))
    batch_p = jnp.concatenate(
        [batch.astype(jnp.int32), jnp.full((_N_PAD - _N,), _NG, jnp.int32)])

    degp = _deg_kernel(src_p, w_p)
    dis = _dis_call(degp).reshape(-1)
    norm_p = _norm_kernel(dis, src_p, dst_p, w_p)

    edata = jnp.concatenate([
        src_p.reshape(_NCHUNKS, _E_BLK),
        dst_p.reshape(_NCHUNKS, _E_BLK),
    ], axis=1).reshape(-1)

    p1 = _prep_cell_params(params['l1'])
    p2 = _prep_cell_params(params['l2'])

    edges = (edata, norm_p, cnts)
    c1 = jnp.zeros((_N_PAD, _H1), jnp.float32)
    c2 = jnp.zeros((_N_PAD, _H2), jnp.float32)
    h1 = h2 = None
    for t in range(t_steps):
        tbl1 = [xp[t]] if t == 0 else [xp[t], h1]
        h1, c1 = _cell(p1, tbl1, edges, c1)
        tbl2 = [h1] if t == 0 else [h1, h2]
        h2, c2 = _cell(p2, tbl2, edges, c2)

    out = _pool_call(h2, batch_p.reshape(_N_PAD, 1),
                     params['lin_W'], params['lin_b'].reshape(1, 1))
    return out.reshape(-1)


# speedup vs baseline: 4.7515x; 1.4258x over previous
import functools

import jax
import jax.numpy as jnp
from jax import lax
from jax.experimental import pallas as pl
from jax.experimental.pallas import tpu as pltpu
from jax.experimental.pallas import tpu_sc as plsc

_K = 5
_N = 10000
_E = 160000
_H1 = 32
_H2 = 16
_NG = 64

_NC = 2
_NS = 16
_NW = _NC * _NS

_N_PAD = 10112
_ROWS_SUB = _N_PAD // _NS
_E_BLK = 128
_E_TILE = 5120
_E_PAD = _E_TILE * _NW
_NH = _N_PAD // 2
_YR = _NH + 64
_RSUB = _YR // _NS
_NCHUNKS = _E_PAD // _E_BLK

_mesh = functools.partial(
    plsc.VectorSubcoreMesh,
    core_axis_name="c", subcore_axis_name="s",
    num_cores=_NC, num_subcores=_NS,
)
_sc_params = pltpu.CompilerParams(use_tc_tiling_on_sc=False)


def _worker():
    cid = lax.axis_index("c")
    sid = lax.axis_index("s")
    return cid, sid, cid * _NS + sid


def _deg_body(src_hbm, w_hbm, out_hbm, wbuf, sbuf, rows, zbuf, deg_sh):
    cid, sid, wid = _worker()

    def zrow(i, _):
        zbuf[i, pl.ds(0, 16)] = jnp.zeros((16,), jnp.float32)
        return 0

    lax.fori_loop(0, _ROWS_SUB, zrow, 0)
    pltpu.sync_copy(zbuf, deg_sh.at[pl.ds(sid * _ROWS_SUB, _ROWS_SUB)])
    plsc.subcore_barrier()

    base = wid * _E_TILE

    def chunk(j, _):
        eb = base + j * _E_BLK
        pltpu.sync_copy(src_hbm.at[pl.ds(eb, _E_BLK)], sbuf.at[0])
        pltpu.sync_copy(w_hbm.at[pl.ds(eb, _E_BLK)], wbuf)

        def build(g, _):
            nv = wbuf[pl.ds(g * 16, 16)]
            for l in range(16):
                rows[g * 16 + l, pl.ds(0, 16)] = jnp.full(
                    (16,), nv[l], jnp.float32)
            return 0

        lax.fori_loop(0, _E_BLK // 16, build, 0)
        pltpu.sync_copy(rows, deg_sh.at[sbuf.at[0]], add=True)
        return 0

    lax.fori_loop(0, _E_TILE // _E_BLK, chunk, 0)
    plsc.subcore_barrier()
    pltpu.sync_copy(deg_sh.at[pl.ds(sid * _ROWS_SUB, _ROWS_SUB)], zbuf)
    pltpu.sync_copy(
        zbuf, out_hbm.at[pl.ds(cid * _N_PAD + sid * _ROWS_SUB, _ROWS_SUB)])


_deg_kernel = pl.kernel(
    _deg_body,
    out_type=jax.ShapeDtypeStruct((_NC * _N_PAD, 16), jnp.float32),
    mesh=_mesh(),
    scratch_types=[
        pltpu.VMEM((_E_BLK,), jnp.float32),
        pltpu.VMEM((1, _E_BLK), jnp.int32),
        pltpu.VMEM((_E_BLK, 16), jnp.float32),
        pltpu.VMEM((_ROWS_SUB, 16), jnp.float32),
        pltpu.VMEM_SHARED((_N_PAD, 16), jnp.float32),
    ],
    compiler_params=_sc_params,
)


def _norm_body(dis_hbm, src_hbm, dst_hbm, w_hbm, out_hbm,
               sbuf, dbuf, wbuf, av, bv, nbuf, sem):
    _, _, wid = _worker()
    base = wid * _E_TILE

    def chunk(j, _):
        eb = base + j * _E_BLK
        pltpu.sync_copy(src_hbm.at[pl.ds(eb, _E_BLK)], sbuf.at[0])
        pltpu.sync_copy(dst_hbm.at[pl.ds(eb, _E_BLK)], dbuf.at[0])
        pltpu.sync_copy(w_hbm.at[pl.ds(eb, _E_BLK)], wbuf)
        pltpu.async_copy(dis_hbm.at[sbuf.at[0]], av, sem).wait()
        pltpu.async_copy(dis_hbm.at[dbuf.at[0]], bv, sem).wait()
        for i in range(_E_BLK // 16):
            s = pl.ds(i * 16, 16)
            nbuf[s] = -av[s] * wbuf[s] * bv[s]
        pltpu.sync_copy(nbuf, out_hbm.at[pl.ds(eb, _E_BLK)])
        return 0

    lax.fori_loop(0, _E_TILE // _E_BLK, chunk, 0)


_norm_kernel = pl.kernel(
    _norm_body,
    out_type=jax.ShapeDtypeStruct((_E_PAD,), jnp.float32),
    mesh=_mesh(),
    scratch_types=[
        pltpu.VMEM((1, _E_BLK), jnp.int32),
        pltpu.VMEM((1, _E_BLK), jnp.int32),
        pltpu.VMEM((_E_BLK,), jnp.float32),
        pltpu.VMEM((_E_BLK,), jnp.float32),
        pltpu.VMEM((_E_BLK,), jnp.float32),
        pltpu.VMEM((_E_BLK,), jnp.float32),
        pltpu.SemaphoreType.DMA,
    ],
    compiler_params=_sc_params,
)


_EW = 2 * _E_BLK
_TROWS = 320
_ACC_R = _TROWS + 8


def _prop_body(d, t_hbm, edata_hbm, norm_hbm, cnts_hbm, out_hbm,
               ebuf, nbuf, lbuf, cbuf, rows, acc, sem_i, sem_g):
    cid, sid, wid = _worker()
    nvec = d // 16

    def zrow(i, _):
        for c in range(nvec):
            acc[i, pl.ds(c * 16, 16)] = jnp.zeros((16,), jnp.float32)
        return 0

    lax.fori_loop(0, _ACC_R, zrow, 0)

    pltpu.sync_copy(cnts_hbm.at[pl.ds(wid * 8, 16)], cbuf)
    cv = cbuf[pl.ds(0, 16)]
    ca = cv[0]
    nk = cv[1] - cv[0]
    base_row = wid * _TROWS

    def body(k, _):
        slot = lax.rem(k, 2)
        pslot = lax.rem(k + 1, 2)
        e3c = lax.rem(k + 1, 3)
        e3p = lax.rem(k + 2, 3)
        e3i = lax.rem(k, 3)

        @pl.when((k >= 1) & (k <= nk))
        def _():
            pltpu.make_async_copy(
                edata_hbm.at[pl.ds(0, _EW)], ebuf.at[e3p], sem_i).wait()
            pltpu.make_async_copy(
                norm_hbm.at[pl.ds(0, _E_BLK)], nbuf.at[e3p], sem_i).wait()
            pltpu.async_copy(
                t_hbm.at[ebuf.at[e3p, pl.ds(0, _E_BLK)]], rows.at[pslot],
                sem_g.at[pslot])

        @pl.when(k < nk)
        def _():
            eo = (ca + k) * _EW
            pltpu.async_copy(
                edata_hbm.at[pl.ds(eo, _EW)], ebuf.at[e3i], sem_i)
            pltpu.async_copy(
                norm_hbm.at[pl.ds((ca + k) * _E_BLK, _E_BLK)], nbuf.at[e3i],
                sem_i)

        @pl.when(k >= 2)
        def _():
            pltpu.make_async_copy(
                t_hbm.at[ebuf.at[e3c, pl.ds(0, _E_BLK)]], rows.at[slot],
                sem_g.at[slot]).wait()

            for g in range(_E_BLK // 16):
                dl = ebuf[e3c, pl.ds(_E_BLK + g * 16, 16)] - base_row
                ok = (dl >= 0) & (dl < _TROWS)
                dummy = _TROWS + (lax.iota(jnp.int32, 16) & 7)
                lbuf[slot, pl.ds(g * 16, 16)] = jnp.where(ok, dl, dummy)

            def accum(g, _):
                nv = nbuf[e3c, pl.ds(g * 16, 16)]
                lv = lbuf[slot, pl.ds(g * 16, 16)]
                for l in range(16):
                    sn = nv[l]
                    li = lv[l]
                    e = g * 16 + l
                    for c in range(nvec):
                        plsc.addupdate(
                            acc.at[li, pl.ds(c * 16, 16)],
                            rows[slot, e, pl.ds(c * 16, 16)] * sn)
                return 0

            lax.fori_loop(0, _E_BLK // 16, accum, 0)

        return 0

    lax.fori_loop(0, nk + 2, body, 0)

    @pl.when(wid < _NW - 1)
    def _():
        pltpu.sync_copy(acc.at[pl.ds(0, _TROWS)],
                        out_hbm.at[pl.ds(base_row, _TROWS)])

    @pl.when(wid == _NW - 1)
    def _():
        nlast = _N_PAD - (_NW - 1) * _TROWS
        pltpu.sync_copy(acc.at[pl.ds(0, nlast)],
                        out_hbm.at[pl.ds(base_row, nlast)])


@functools.cache
def _prop_kernel(d):
    return pl.kernel(
        functools.partial(_prop_body, d),
        out_type=jax.ShapeDtypeStruct((_N_PAD, d), jnp.float32),
        mesh=_mesh(),
        scratch_types=[
            pltpu.VMEM((3, _EW), jnp.int32),
            pltpu.VMEM((3, _E_BLK), jnp.float32),
            pltpu.VMEM((2, _E_BLK), jnp.int32),
            pltpu.VMEM((16,), jnp.int32),
            pltpu.VMEM((2, _E_BLK, d), jnp.float32),
            pltpu.VMEM((_ACC_R, d), jnp.float32),
            pltpu.SemaphoreType.DMA,
            pltpu.SemaphoreType.DMA((2,)),
        ],
        compiler_params=_sc_params,
    )


_RB = _N_PAD // 8


def _dis_body(degp_ref, out_ref):
    deg = degp_ref[0:_N_PAD, 0:1] + degp_ref[_N_PAD:2 * _N_PAD, 0:1]
    out_ref[...] = jnp.where(
        deg > 0.0, lax.rsqrt(jnp.where(deg > 0.0, deg, 1.0)), 0.0)


def _dis_call(degp):
    return pl.pallas_call(
        _dis_body,
        out_shape=jax.ShapeDtypeStruct((_N_PAD, 1), jnp.float32),
    )(degp)


def _combine_body(y_ref, p_ref, out_ref):
    out_ref[...] = 2.0 * y_ref[...] - p_ref[...]


@functools.cache
def _combine_call(d):
    spec = pl.BlockSpec((_RB, d), lambda i: (i, 0))
    return pl.pallas_call(
        _combine_body,
        grid=(8,),
        in_specs=[spec, spec],
        out_specs=spec,
        out_shape=jax.ShapeDtypeStruct((_N_PAD, d), jnp.float32),
    )


def _gates_body(oc, nb, *refs):
    b_refs = refs[:nb]
    w_refs = refs[nb:2 * nb]
    bias_ref, wc_ref, c_ref, h_out, c_out = refs[2 * nb:]
    bias = (bias_ref[0:1, :] + bias_ref[1:2, :] + bias_ref[2:3, :])
    z = jnp.broadcast_to(bias, (_RB, 4 * oc)).astype(jnp.float32)
    for b_ref, w_ref in zip(b_refs, w_refs):
        for k in range(_K):
            z = z + jnp.dot(b_ref[k], w_ref[k],
                            preferred_element_type=jnp.float32)
    c_prev = c_ref[...]
    i_g = jax.nn.sigmoid(z[:, 0 * oc:1 * oc] + wc_ref[0:1, :] * c_prev)
    f_g = jax.nn.sigmoid(z[:, 1 * oc:2 * oc] + wc_ref[1:2, :] * c_prev)
    t_g = jnp.tanh(z[:, 2 * oc:3 * oc])
    c_new = f_g * c_prev + i_g * t_g
    o_g = jax.nn.sigmoid(z[:, 3 * oc:4 * oc] + wc_ref[2:3, :] * c_new)
    h_out[...] = o_g * jnp.tanh(c_new)
    c_out[...] = c_new


@functools.cache
def _gates_call(ds, oc):
    return pl.pallas_call(
        functools.partial(_gates_body, oc, len(ds)),
        grid=(8,),
        in_specs=(
            [pl.BlockSpec((_K, _RB, d), lambda i: (0, i, 0)) for d in ds]
            + [pl.BlockSpec((_K, d, 4 * oc), lambda i: (0, 0, 0)) for d in ds]
            + [
                pl.BlockSpec((3, 4 * oc), lambda i: (0, 0)),
                pl.BlockSpec((3, oc), lambda i: (0, 0)),
                pl.BlockSpec((_RB, oc), lambda i: (i, 0)),
            ]
        ),
        out_specs=[
            pl.BlockSpec((_RB, oc), lambda i: (i, 0)),
            pl.BlockSpec((_RB, oc), lambda i: (i, 0)),
        ],
        out_shape=[
            jax.ShapeDtypeStruct((_N_PAD, oc), jnp.float32),
            jax.ShapeDtypeStruct((_N_PAD, oc), jnp.float32),
        ],
    )


def _pool_body(h2_ref, b_ref, lw_ref, lb_ref, out_ref, u_scr):
    i = pl.program_id(0)

    @pl.when(i == 0)
    def _():
        u_scr[...] = jnp.zeros_like(u_scr)

    oh = (b_ref[...] == lax.broadcasted_iota(jnp.int32, (_RB, _NG), 1))
    oh = oh.astype(jnp.float32)
    u_scr[...] += lax.dot_general(
        oh, h2_ref[...], (((0,), (0,)), ((), ())),
        preferred_element_type=jnp.float32)

    @pl.when(i == pl.num_programs(0) - 1)
    def _():
        out_ref[...] = (jnp.dot(u_scr[...], lw_ref[...],
                                preferred_element_type=jnp.float32)
                        + lb_ref[...])


_pool_call = pl.pallas_call(
    _pool_body,
    grid=(8,),
    in_specs=[
        pl.BlockSpec((_RB, _H2), lambda i: (i, 0)),
        pl.BlockSpec((_RB, 1), lambda i: (i, 0)),
        pl.BlockSpec((_H2, 1), lambda i: (0, 0)),
        pl.BlockSpec((1, 1), lambda i: (0, 0)),
    ],
    out_specs=pl.BlockSpec((_NG, 1), lambda i: (0, 0)),
    out_shape=jax.ShapeDtypeStruct((_NG, 1), jnp.float32),
    scratch_shapes=[pltpu.VMEM((_NG, _H2), jnp.float32)],
)


def _basis(table, edata, norm_p, cnts):
    d = table.shape[1]
    prop = _prop_kernel(d)
    comb = _combine_call(d)
    txs = [table, prop(table, edata, norm_p, cnts)]
    for _ in range(2, _K):
        y = prop(txs[-1], edata, norm_p, cnts)
        txs.append(comb(y, txs[-2]))
    return jnp.stack(txs)


def _cell(pcat, tables, edges, c_prev):
    wcat, bias3, wc = pcat
    oc = wc.shape[1]
    bs, ws, off = [], [], 0
    for t in tables:
        d = t.shape[1]
        bs.append(_basis(t, *edges))
        ws.append(wcat[:, off:off + d, :])
        off += d
    ds = tuple(t.shape[1] for t in tables)
    h, c = _gates_call(ds, oc)(*bs, *ws, bias3, wc, c_prev)
    return h, c


def _prep_cell_params(p):
    wx = jnp.concatenate([p['Wx_' + g] for g in 'ifco'], axis=2)
    wh = jnp.concatenate([p['Wh_' + g] for g in 'ifco'], axis=2)
    wcat = jnp.concatenate([wx, wh], axis=1)
    bias3 = jnp.stack([
        jnp.concatenate([p['bx_' + g] for g in 'ifco']),
        jnp.concatenate([p['bh_' + g] for g in 'ifco']),
        jnp.concatenate([p['b_' + g] for g in 'ifco']),
    ])
    wc = jnp.stack([p['w_c_i'], p['w_c_f'], p['w_c_o']])
    return wcat, bias3, wc


def kernel(x, edge_index, edge_weight, batch, params):
    t_steps = x.shape[0]
    src = edge_index[0].astype(jnp.int32)
    dst = edge_index[1].astype(jnp.int32)

    order = jnp.argsort(dst)
    dst_s = dst[order]
    npad_e = _E_PAD - _E
    src_p = jnp.concatenate([src[order], jnp.zeros((npad_e,), jnp.int32)])
    dst_p = jnp.concatenate([dst_s, jnp.full((npad_e,), _N, jnp.int32)])
    w_p = jnp.concatenate(
        [edge_weight[order], jnp.zeros((npad_e,), jnp.float32)])
    tb = jnp.arange(_NW + 1, dtype=jnp.int32) * _TROWS
    pos = jnp.searchsorted(dst_s, tb, side='left').astype(jnp.int32)
    ca = pos[:-1] // _E_BLK
    cb = (pos[1:] + _E_BLK - 1) // _E_BLK
    cnts = jnp.zeros((_NW * 8 + 8,), jnp.int32)
    cnts = cnts.at[jnp.arange(_NW) * 8].set(ca)
    cnts = cnts.at[jnp.arange(_NW) * 8 + 1].set(cb)

    xp = jnp.pad(x, ((0, 0), (0, _N_PAD - _N), (0, 0)))
    batch_p = jnp.concatenate(
        [batch.astype(jnp.int32), jnp.full((_N_PAD - _N,), _NG, jnp.int32)])

    degp = _deg_kernel(src_p, w_p)
    dis = _dis_call(degp).reshape(-1)
    norm_p = _norm_kernel(dis, src_p, dst_p, w_p)

    edata = jnp.concatenate([
        src_p.reshape(_NCHUNKS, _E_BLK),
        dst_p.reshape(_NCHUNKS, _E_BLK),
    ], axis=1).reshape(-1)

    p1 = _prep_cell_params(params['l1'])
    p2 = _prep_cell_params(params['l2'])

    edges = (edata, norm_p, cnts)
    c1 = jnp.zeros((_N_PAD, _H1), jnp.float32)
    c2 = jnp.zeros((_N_PAD, _H2), jnp.float32)
    h1 = h2 = None
    for t in range(t_steps):
        tbl1 = [xp[t]] if t == 0 else [xp[t], h1]
        h1, c1 = _cell(p1, tbl1, edges, c1)
        tbl2 = [h1] if t == 0 else [h1, h2]
        h2, c2 = _cell(p2, tbl2, edges, c2)

    out = _pool_call(h2, batch_p.reshape(_N_PAD, 1),
                     params['lin_W'], params['lin_b'].reshape(1, 1))
    return out.reshape(-1)


# speedup vs baseline: 6.6911x; 1.4082x over previous
import functools

import jax
import jax.numpy as jnp
from jax import lax
from jax.experimental import pallas as pl
from jax.experimental.pallas import tpu as pltpu
from jax.experimental.pallas import tpu_sc as plsc

_K = 5
_N = 10000
_E = 160000
_H1 = 32
_H2 = 16
_NG = 64

_NC = 2
_NS = 16
_NW = _NC * _NS

_N_PAD = 10112
_ROWS_SUB = _N_PAD // _NS
_E_BLK = 128
_E_TILE = 5120
_E_PAD = _E_TILE * _NW
_NH = _N_PAD // 2
_YR = _NH + 64
_RSUB = _YR // _NS
_NCHUNKS = _E_PAD // _E_BLK

_mesh = functools.partial(
    plsc.VectorSubcoreMesh,
    core_axis_name="c", subcore_axis_name="s",
    num_cores=_NC, num_subcores=_NS,
)
_sc_params = pltpu.CompilerParams(use_tc_tiling_on_sc=False)


def _worker():
    cid = lax.axis_index("c")
    sid = lax.axis_index("s")
    return cid, sid, cid * _NS + sid


def _deg_body(src_hbm, w_hbm, out_hbm, wbuf, sbuf, rows, zbuf, deg_sh):
    cid, sid, wid = _worker()

    def zrow(i, _):
        zbuf[i, pl.ds(0, 16)] = jnp.zeros((16,), jnp.float32)
        return 0

    lax.fori_loop(0, _ROWS_SUB, zrow, 0)
    pltpu.sync_copy(zbuf, deg_sh.at[pl.ds(sid * _ROWS_SUB, _ROWS_SUB)])
    plsc.subcore_barrier()

    base = wid * _E_TILE

    def chunk(j, _):
        eb = base + j * _E_BLK
        pltpu.sync_copy(src_hbm.at[pl.ds(eb, _E_BLK)], sbuf.at[0])
        pltpu.sync_copy(w_hbm.at[pl.ds(eb, _E_BLK)], wbuf)

        def build(g, _):
            nv = wbuf[pl.ds(g * 16, 16)]
            for l in range(16):
                rows[g * 16 + l, pl.ds(0, 16)] = jnp.full(
                    (16,), nv[l], jnp.float32)
            return 0

        lax.fori_loop(0, _E_BLK // 16, build, 0)
        pltpu.sync_copy(rows, deg_sh.at[sbuf.at[0]], add=True)
        return 0

    lax.fori_loop(0, _E_TILE // _E_BLK, chunk, 0)
    plsc.subcore_barrier()
    pltpu.sync_copy(deg_sh.at[pl.ds(sid * _ROWS_SUB, _ROWS_SUB)], zbuf)
    pltpu.sync_copy(
        zbuf, out_hbm.at[pl.ds(cid * _N_PAD + sid * _ROWS_SUB, _ROWS_SUB)])


_deg_kernel = pl.kernel(
    _deg_body,
    out_type=jax.ShapeDtypeStruct((_NC * _N_PAD, 16), jnp.float32),
    mesh=_mesh(),
    scratch_types=[
        pltpu.VMEM((_E_BLK,), jnp.float32),
        pltpu.VMEM((1, _E_BLK), jnp.int32),
        pltpu.VMEM((_E_BLK, 16), jnp.float32),
        pltpu.VMEM((_ROWS_SUB, 16), jnp.float32),
        pltpu.VMEM_SHARED((_N_PAD, 16), jnp.float32),
    ],
    compiler_params=_sc_params,
)


def _norm_body(dis_hbm, src_hbm, dst_hbm, w_hbm, out_hbm,
               sbuf, dbuf, wbuf, av, bv, nbuf, sem):
    _, _, wid = _worker()
    base = wid * _E_TILE

    def chunk(j, _):
        eb = base + j * _E_BLK
        pltpu.sync_copy(src_hbm.at[pl.ds(eb, _E_BLK)], sbuf.at[0])
        pltpu.sync_copy(dst_hbm.at[pl.ds(eb, _E_BLK)], dbuf.at[0])
        pltpu.sync_copy(w_hbm.at[pl.ds(eb, _E_BLK)], wbuf)
        pltpu.async_copy(dis_hbm.at[sbuf.at[0]], av, sem).wait()
        pltpu.async_copy(dis_hbm.at[dbuf.at[0]], bv, sem).wait()
        for i in range(_E_BLK // 16):
            s = pl.ds(i * 16, 16)
            nbuf[s] = -av[s] * wbuf[s] * bv[s]
        pltpu.sync_copy(nbuf, out_hbm.at[pl.ds(eb, _E_BLK)])
        return 0

    lax.fori_loop(0, _E_TILE // _E_BLK, chunk, 0)


_norm_kernel = pl.kernel(
    _norm_body,
    out_type=jax.ShapeDtypeStruct((_E_PAD,), jnp.float32),
    mesh=_mesh(),
    scratch_types=[
        pltpu.VMEM((1, _E_BLK), jnp.int32),
        pltpu.VMEM((1, _E_BLK), jnp.int32),
        pltpu.VMEM((_E_BLK,), jnp.float32),
        pltpu.VMEM((_E_BLK,), jnp.float32),
        pltpu.VMEM((_E_BLK,), jnp.float32),
        pltpu.VMEM((_E_BLK,), jnp.float32),
        pltpu.SemaphoreType.DMA,
    ],
    compiler_params=_sc_params,
)


_EW = 2 * _E_BLK
_TROWS = 320
_ACC_R = _TROWS + 8


def _prop_body(d, t_hbm, edata_hbm, norm_hbm, cnts_hbm, out_hbm,
               ebuf, nbuf, lbuf, cbuf, rows, acc, sem_i, sem_g):
    cid, sid, wid = _worker()
    nvec = d // 16

    def zrow(i, _):
        for c in range(nvec):
            acc[i, pl.ds(c * 16, 16)] = jnp.zeros((16,), jnp.float32)
        return 0

    lax.fori_loop(0, _ACC_R, zrow, 0)

    pltpu.sync_copy(cnts_hbm.at[pl.ds(wid * 8, 16)], cbuf)
    cv = cbuf[pl.ds(0, 16)]
    ca = cv[0]
    nk = cv[1] - cv[0]
    base_row = wid * _TROWS

    def body(k, _):
        slot = lax.rem(k, 2)
        pslot = lax.rem(k + 1, 2)
        e3c = lax.rem(k + 1, 3)
        e3p = lax.rem(k + 2, 3)
        e3i = lax.rem(k, 3)

        @pl.when((k >= 1) & (k <= nk))
        def _():
            pltpu.make_async_copy(
                edata_hbm.at[pl.ds(0, _EW)], ebuf.at[e3p], sem_i).wait()
            pltpu.make_async_copy(
                norm_hbm.at[pl.ds(0, _E_BLK)], nbuf.at[e3p], sem_i).wait()
            pltpu.async_copy(
                t_hbm.at[ebuf.at[e3p, pl.ds(0, _E_BLK)]], rows.at[pslot],
                sem_g.at[pslot])

        @pl.when(k < nk)
        def _():
            eo = (ca + k) * _EW
            pltpu.async_copy(
                edata_hbm.at[pl.ds(eo, _EW)], ebuf.at[e3i], sem_i)
            pltpu.async_copy(
                norm_hbm.at[pl.ds((ca + k) * _E_BLK, _E_BLK)], nbuf.at[e3i],
                sem_i)

        @pl.when(k >= 2)
        def _():
            pltpu.make_async_copy(
                t_hbm.at[ebuf.at[e3c, pl.ds(0, _E_BLK)]], rows.at[slot],
                sem_g.at[slot]).wait()

            for g in range(_E_BLK // 16):
                dl = ebuf[e3c, pl.ds(_E_BLK + g * 16, 16)] - base_row
                ok = (dl >= 0) & (dl < _TROWS)
                dummy = _TROWS + (lax.iota(jnp.int32, 16) & 7)
                lbuf[slot, pl.ds(g * 16, 16)] = jnp.where(ok, dl, dummy)

            @plsc.parallel_loop(0, _E_BLK // 16, unroll=2)
            def accum(g):
                nv = nbuf[e3c, pl.ds(g * 16, 16)]
                lv = lbuf[slot, pl.ds(g * 16, 16)]
                for l in range(16):
                    sn = nv[l]
                    li = lv[l]
                    e = g * 16 + l
                    for c in range(nvec):
                        plsc.addupdate(
                            acc.at[li, pl.ds(c * 16, 16)],
                            rows[slot, e, pl.ds(c * 16, 16)] * sn)

        return 0

    lax.fori_loop(0, nk + 2, body, 0)

    @pl.when(wid < _NW - 1)
    def _():
        pltpu.sync_copy(acc.at[pl.ds(0, _TROWS)],
                        out_hbm.at[pl.ds(base_row, _TROWS)])

    @pl.when(wid == _NW - 1)
    def _():
        nlast = _N_PAD - (_NW - 1) * _TROWS
        pltpu.sync_copy(acc.at[pl.ds(0, nlast)],
                        out_hbm.at[pl.ds(base_row, nlast)])


@functools.cache
def _prop_kernel(d):
    return pl.kernel(
        functools.partial(_prop_body, d),
        out_type=jax.ShapeDtypeStruct((_N_PAD, d), jnp.float32),
        mesh=_mesh(),
        scratch_types=[
            pltpu.VMEM((3, _EW), jnp.int32),
            pltpu.VMEM((3, _E_BLK), jnp.float32),
            pltpu.VMEM((2, _E_BLK), jnp.int32),
            pltpu.VMEM((16,), jnp.int32),
            pltpu.VMEM((2, _E_BLK, d), jnp.float32),
            pltpu.VMEM((_ACC_R, d), jnp.float32),
            pltpu.SemaphoreType.DMA,
            pltpu.SemaphoreType.DMA((2,)),
        ],
        compiler_params=_sc_params,
    )


_RB = _N_PAD // 8


def _dis_body(degp_ref, out_ref):
    deg = degp_ref[0:_N_PAD, 0:1] + degp_ref[_N_PAD:2 * _N_PAD, 0:1]
    out_ref[...] = jnp.where(
        deg > 0.0, lax.rsqrt(jnp.where(deg > 0.0, deg, 1.0)), 0.0)


def _dis_call(degp):
    return pl.pallas_call(
        _dis_body,
        out_shape=jax.ShapeDtypeStruct((_N_PAD, 1), jnp.float32),
    )(degp)


def _combine_body(y_ref, p_ref, out_ref):
    out_ref[...] = 2.0 * y_ref[...] - p_ref[...]


@functools.cache
def _combine_call(d):
    spec = pl.BlockSpec((_RB, d), lambda i: (i, 0))
    return pl.pallas_call(
        _combine_body,
        grid=(8,),
        in_specs=[spec, spec],
        out_specs=spec,
        out_shape=jax.ShapeDtypeStruct((_N_PAD, d), jnp.float32),
    )


def _gates_body(oc, nb, *refs):
    b_refs = refs[:nb]
    w_refs = refs[nb:2 * nb]
    bias_ref, wc_ref, c_ref, h_out, c_out = refs[2 * nb:]
    bias = (bias_ref[0:1, :] + bias_ref[1:2, :] + bias_ref[2:3, :])
    z = jnp.broadcast_to(bias, (_RB, 4 * oc)).astype(jnp.float32)
    for b_ref, w_ref in zip(b_refs, w_refs):
        for k in range(_K):
            z = z + jnp.dot(b_ref[k], w_ref[k],
                            preferred_element_type=jnp.float32)
    c_prev = c_ref[...]
    i_g = jax.nn.sigmoid(z[:, 0 * oc:1 * oc] + wc_ref[0:1, :] * c_prev)
    f_g = jax.nn.sigmoid(z[:, 1 * oc:2 * oc] + wc_ref[1:2, :] * c_prev)
    t_g = jnp.tanh(z[:, 2 * oc:3 * oc])
    c_new = f_g * c_prev + i_g * t_g
    o_g = jax.nn.sigmoid(z[:, 3 * oc:4 * oc] + wc_ref[2:3, :] * c_new)
    h_out[...] = o_g * jnp.tanh(c_new)
    c_out[...] = c_new


@functools.cache
def _gates_call(ds, oc):
    return pl.pallas_call(
        functools.partial(_gates_body, oc, len(ds)),
        grid=(8,),
        in_specs=(
            [pl.BlockSpec((_K, _RB, d), lambda i: (0, i, 0)) for d in ds]
            + [pl.BlockSpec((_K, d, 4 * oc), lambda i: (0, 0, 0)) for d in ds]
            + [
                pl.BlockSpec((3, 4 * oc), lambda i: (0, 0)),
                pl.BlockSpec((3, oc), lambda i: (0, 0)),
                pl.BlockSpec((_RB, oc), lambda i: (i, 0)),
            ]
        ),
        out_specs=[
            pl.BlockSpec((_RB, oc), lambda i: (i, 0)),
            pl.BlockSpec((_RB, oc), lambda i: (i, 0)),
        ],
        out_shape=[
            jax.ShapeDtypeStruct((_N_PAD, oc), jnp.float32),
            jax.ShapeDtypeStruct((_N_PAD, oc), jnp.float32),
        ],
    )


def _pool_body(h2_ref, b_ref, lw_ref, lb_ref, out_ref, u_scr):
    i = pl.program_id(0)

    @pl.when(i == 0)
    def _():
        u_scr[...] = jnp.zeros_like(u_scr)

    oh = (b_ref[...] == lax.broadcasted_iota(jnp.int32, (_RB, _NG), 1))
    oh = oh.astype(jnp.float32)
    u_scr[...] += lax.dot_general(
        oh, h2_ref[...], (((0,), (0,)), ((), ())),
        preferred_element_type=jnp.float32)

    @pl.when(i == pl.num_programs(0) - 1)
    def _():
        out_ref[...] = (jnp.dot(u_scr[...], lw_ref[...],
                                preferred_element_type=jnp.float32)
                        + lb_ref[...])


_pool_call = pl.pallas_call(
    _pool_body,
    grid=(8,),
    in_specs=[
        pl.BlockSpec((_RB, _H2), lambda i: (i, 0)),
        pl.BlockSpec((_RB, 1), lambda i: (i, 0)),
        pl.BlockSpec((_H2, 1), lambda i: (0, 0)),
        pl.BlockSpec((1, 1), lambda i: (0, 0)),
    ],
    out_specs=pl.BlockSpec((_NG, 1), lambda i: (0, 0)),
    out_shape=jax.ShapeDtypeStruct((_NG, 1), jnp.float32),
    scratch_shapes=[pltpu.VMEM((_NG, _H2), jnp.float32)],
)


def _basis(table, edata, norm_p, cnts):
    d = table.shape[1]
    prop = _prop_kernel(d)
    comb = _combine_call(d)
    txs = [table, prop(table, edata, norm_p, cnts)]
    for _ in range(2, _K):
        y = prop(txs[-1], edata, norm_p, cnts)
        txs.append(comb(y, txs[-2]))
    return jnp.stack(txs)


def _cell(pcat, tables, edges, c_prev):
    wcat, bias3, wc = pcat
    oc = wc.shape[1]
    bs, ws, off = [], [], 0
    for t in tables:
        d = t.shape[1]
        bs.append(_basis(t, *edges))
        ws.append(wcat[:, off:off + d, :])
        off += d
    ds = tuple(t.shape[1] for t in tables)
    h, c = _gates_call(ds, oc)(*bs, *ws, bias3, wc, c_prev)
    return h, c


def _prep_cell_params(p):
    wx = jnp.concatenate([p['Wx_' + g] for g in 'ifco'], axis=2)
    wh = jnp.concatenate([p['Wh_' + g] for g in 'ifco'], axis=2)
    wcat = jnp.concatenate([wx, wh], axis=1)
    bias3 = jnp.stack([
        jnp.concatenate([p['bx_' + g] for g in 'ifco']),
        jnp.concatenate([p['bh_' + g] for g in 'ifco']),
        jnp.concatenate([p['b_' + g] for g in 'ifco']),
    ])
    wc = jnp.stack([p['w_c_i'], p['w_c_f'], p['w_c_o']])
    return wcat, bias3, wc


def kernel(x, edge_index, edge_weight, batch, params):
    t_steps = x.shape[0]
    src = edge_index[0].astype(jnp.int32)
    dst = edge_index[1].astype(jnp.int32)

    order = jnp.argsort(dst)
    dst_s = dst[order]
    npad_e = _E_PAD - _E
    src_p = jnp.concatenate([src[order], jnp.zeros((npad_e,), jnp.int32)])
    dst_p = jnp.concatenate([dst_s, jnp.full((npad_e,), _N, jnp.int32)])
    w_p = jnp.concatenate(
        [edge_weight[order], jnp.zeros((npad_e,), jnp.float32)])
    tb = jnp.arange(_NW + 1, dtype=jnp.int32) * _TROWS
    pos = jnp.searchsorted(dst_s, tb, side='left').astype(jnp.int32)
    ca = pos[:-1] // _E_BLK
    cb = (pos[1:] + _E_BLK - 1) // _E_BLK
    cnts = jnp.zeros((_NW * 8 + 8,), jnp.int32)
    cnts = cnts.at[jnp.arange(_NW) * 8].set(ca)
    cnts = cnts.at[jnp.arange(_NW) * 8 + 1].set(cb)

    xp = jnp.pad(x, ((0, 0), (0, _N_PAD - _N), (0, 0)))
    batch_p = jnp.concatenate(
        [batch.astype(jnp.int32), jnp.full((_N_PAD - _N,), _NG, jnp.int32)])

    degp = _deg_kernel(src_p, w_p)
    dis = _dis_call(degp).reshape(-1)
    norm_p = _norm_kernel(dis, src_p, dst_p, w_p)

    edata = jnp.concatenate([
        src_p.reshape(_NCHUNKS, _E_BLK),
        dst_p.reshape(_NCHUNKS, _E_BLK),
    ], axis=1).reshape(-1)

    p1 = _prep_cell_params(params['l1'])
    p2 = _prep_cell_params(params['l2'])

    edges = (edata, norm_p, cnts)
    c1 = jnp.zeros((_N_PAD, _H1), jnp.float32)
    c2 = jnp.zeros((_N_PAD, _H2), jnp.float32)
    h1 = h2 = None
    for t in range(t_steps):
        tbl1 = [xp[t]] if t == 0 else [xp[t], h1]
        h1, c1 = _cell(p1, tbl1, edges, c1)
        tbl2 = [h1] if t == 0 else [h1, h2]
        h2, c2 = _cell(p2, tbl2, edges, c2)

    out = _pool_call(h2, batch_p.reshape(_N_PAD, 1),
                     params['lin_W'], params['lin_b'].reshape(1, 1))
    return out.reshape(-1)
